# baseline (speedup 1.0000x reference)
import functools

import jax
import jax.numpy as jnp
from jax import lax
from jax.experimental import pallas as pl
from jax.experimental.pallas import tpu as pltpu
from jax.experimental.pallas import tpu_sc as plsc

HIDDEN = 768
LANES = 16
NVEC = HIDDEN // LANES
TOK = 8192
NC = 2
NS = 16
NW = NC * NS
PER_W = TOK // NW
CH = 32
NCH = PER_W // CH
LN_EPS = 1e-12


_GATHER_DNUMS = lax.GatherDimensionNumbers(
    offset_dims=(), collapsed_slice_dims=(0,), start_index_map=(0,))


def _shuffle(x, idx):
    return lax.gather(x, idx[:, None], _GATHER_DNUMS, slice_sizes=(1,),
                      mode=lax.GatherScatterMode.PROMISE_IN_BOUNDS)


def _sum_lanes(x):
    lanes = lax.iota(jnp.int32, LANES)
    for k in (8, 4, 2, 1):
        x = x + _shuffle(x, lanes ^ k)
    return x


def _rsqrt_vec(x):
    i = lax.bitcast_convert_type(x, jnp.int32)
    i = jnp.int32(0x5F3759DF) - lax.shift_right_logical(i, 1)
    y = lax.bitcast_convert_type(i, jnp.float32)
    for _ in range(3):
        y = y * (1.5 - 0.5 * x * y * y)
    return y


_mesh = plsc.VectorSubcoreMesh(core_axis_name="c", subcore_axis_name="s")


@functools.partial(
    pl.kernel,
    mesh=_mesh,
    out_type=jax.ShapeDtypeStruct((TOK, HIDDEN), jnp.float32),
    scratch_types=[
        pltpu.VMEM((PER_W,), jnp.int32),
        pltpu.VMEM((PER_W,), jnp.int32),
        pltpu.VMEM((PER_W,), jnp.int32),
        pltpu.VMEM((CH, HIDDEN), jnp.float32),
        pltpu.VMEM((CH, HIDDEN), jnp.float32),
        pltpu.VMEM((CH, HIDDEN), jnp.float32),
        pltpu.VMEM((HIDDEN,), jnp.float32),
        pltpu.VMEM((HIDDEN,), jnp.float32),
        pltpu.SemaphoreType.DMA,
        pltpu.SemaphoreType.DMA,
        pltpu.SemaphoreType.DMA,
    ],
)
def _embed_ln(ids_h, pids_h, tids_h, wtab_h, ptab_h, ttab_h, g_h, b_h,
              out_h, idw, idp, idt, bw, bp, bt, gv, bv, s0, s1, s2):
    wid = lax.axis_index("s") * NC + lax.axis_index("c")
    base = wid * PER_W
    pltpu.sync_copy(ids_h.at[pl.ds(base, PER_W)], idw)
    pltpu.sync_copy(pids_h.at[pl.ds(base, PER_W)], idp)
    pltpu.sync_copy(tids_h.at[pl.ds(base, PER_W)], idt)
    pltpu.sync_copy(g_h, gv)
    pltpu.sync_copy(b_h, bv)

    for c in range(NCH):
        off = c * CH
        d0 = pltpu.async_copy(wtab_h.at[idw.at[pl.ds(off, CH)]], bw, s0)
        d1 = pltpu.async_copy(ptab_h.at[idp.at[pl.ds(off, CH)]], bp, s1)
        d2 = pltpu.async_copy(ttab_h.at[idt.at[pl.ds(off, CH)]], bt, s2)
        d0.wait()
        d1.wait()
        d2.wait()

        def token_body(t, _):
            def p1(j, carry):
                acc, acc2 = carry
                sl = pl.ds(j * LANES, LANES)
                s = bw[t, sl] + bp[t, sl] + bt[t, sl]
                bw[t, sl] = s
                return acc + s, acc2 + s * s

            zero = jnp.zeros((LANES,), jnp.float32)
            acc, acc2 = lax.fori_loop(0, NVEC, p1, (zero, zero))
            mean = _sum_lanes(acc) * (1.0 / HIDDEN)
            var = _sum_lanes(acc2) * (1.0 / HIDDEN) - mean * mean
            rstd = _rsqrt_vec(var + LN_EPS)
            shift = mean * rstd

            def p2(j, _2):
                sl = pl.ds(j * LANES, LANES)
                s = bw[t, sl]
                o = (s * rstd - shift) * gv[sl] + bv[sl]
                bw[t, sl] = o
                return 0

            lax.fori_loop(0, NVEC, p2, 0)
            return 0

        lax.fori_loop(0, CH, token_body, 0)
        pltpu.sync_copy(bw, out_h.at[pl.ds(base + off, CH)])


def kernel(input_ids, token_type_ids, position_ids, word_embeddings,
           position_embeddings, token_type_embeddings, ln_gamma, ln_beta):
    B, S = input_ids.shape
    ids = jnp.asarray(input_ids, jnp.int32).reshape(-1)
    tids = jnp.asarray(token_type_ids, jnp.int32).reshape(-1)
    pids = jnp.asarray(position_ids, jnp.int32).reshape(-1)
    out = _embed_ln(ids, pids, tids, word_embeddings, position_embeddings,
                    token_type_embeddings, ln_gamma, ln_beta)
    return out.reshape(B, S, HIDDEN)


# speedup vs baseline: 1.3429x; 1.3429x over previous
import functools

import jax
import jax.numpy as jnp
from jax import lax
from jax.experimental import pallas as pl
from jax.experimental.pallas import tpu as pltpu
from jax.experimental.pallas import tpu_sc as plsc

HIDDEN = 768
LANES = 16
NVEC = HIDDEN // LANES
TOK = 8192
NC = 2
NS = 16
NW = NC * NS
PER_W = TOK // NW
CH = 16
NCH = PER_W // CH
NPAIR = NCH // 2
TG = 4
LN_EPS = 1e-12

_GATHER_DNUMS = lax.GatherDimensionNumbers(
    offset_dims=(), collapsed_slice_dims=(0,), start_index_map=(0,))


def _shuffle(x, idx):
    return lax.gather(x, idx[:, None], _GATHER_DNUMS, slice_sizes=(1,),
                      mode=lax.GatherScatterMode.PROMISE_IN_BOUNDS)


def _sum_lanes(x):
    lanes = lax.iota(jnp.int32, LANES)
    for k in (8, 4, 2, 1):
        x = x + _shuffle(x, lanes ^ k)
    return x


def _rsqrt_vec(x):
    i = lax.bitcast_convert_type(x, jnp.int32)
    i = jnp.int32(0x5F3759DF) - lax.shift_right_logical(i, 1)
    y = lax.bitcast_convert_type(i, jnp.float32)
    for _ in range(3):
        y = y * (1.5 - 0.5 * x * y * y)
    return y


_mesh = plsc.VectorSubcoreMesh(core_axis_name="c", subcore_axis_name="s")


@functools.partial(
    pl.kernel,
    mesh=_mesh,
    out_type=jax.ShapeDtypeStruct((TOK, HIDDEN), jnp.float32),
    scratch_types=[
        pltpu.VMEM((PER_W,), jnp.int32),
        pltpu.VMEM((PER_W,), jnp.int32),
        pltpu.VMEM((PER_W,), jnp.int32),
        pltpu.VMEM((CH, HIDDEN), jnp.float32),
        pltpu.VMEM((CH, HIDDEN), jnp.float32),
        pltpu.VMEM((CH, HIDDEN), jnp.float32),
        pltpu.VMEM((CH, HIDDEN), jnp.float32),
        pltpu.VMEM((CH, HIDDEN), jnp.float32),
        pltpu.VMEM((CH, HIDDEN), jnp.float32),
        pltpu.VMEM((CH, HIDDEN), jnp.float32),
        pltpu.VMEM((CH, HIDDEN), jnp.float32),
        pltpu.VMEM((CH, LANES), jnp.float32),
        pltpu.VMEM((CH, LANES), jnp.float32),
        pltpu.VMEM((HIDDEN,), jnp.float32),
        pltpu.VMEM((HIDDEN,), jnp.float32),
        pltpu.SemaphoreType.DMA,
        pltpu.SemaphoreType.DMA,
        pltpu.SemaphoreType.DMA,
        pltpu.SemaphoreType.DMA,
        pltpu.SemaphoreType.DMA,
        pltpu.SemaphoreType.DMA,
        pltpu.SemaphoreType.DMA,
        pltpu.SemaphoreType.DMA,
    ],
)
def _embed_ln(ids_h, pids_h, tids_h, wtab_h, ptab_h, ttab_h, g_h, b_h,
              out_h, idw, idp, idt,
              bw0, bp0, bt0, bw1, bp1, bt1, ow0, ow1, rs_buf, nsh_buf,
              gv, bv, sw0, sp0, st0, sw1, sp1, st1, so0, so1):
    wid = lax.axis_index("s") * NC + lax.axis_index("c")
    base = wid * PER_W
    pltpu.sync_copy(ids_h.at[pl.ds(base, PER_W)], idw)
    pltpu.sync_copy(pids_h.at[pl.ds(base, PER_W)], idp)
    pltpu.sync_copy(tids_h.at[pl.ds(base, PER_W)], idt)
    pltpu.sync_copy(g_h, gv)
    pltpu.sync_copy(b_h, bv)

    slots = (
        (bw0, bp0, bt0, ow0, sw0, sp0, st0, so0),
        (bw1, bp1, bt1, ow1, sw1, sp1, st1, so1),
    )

    def gathers(off, bw, bp, bt, sw, sp, st):
        return (
            pltpu.make_async_copy(wtab_h.at[idw.at[pl.ds(off, CH)]], bw, sw),
            pltpu.make_async_copy(ptab_h.at[idp.at[pl.ds(off, CH)]], bp, sp),
            pltpu.make_async_copy(ttab_h.at[idt.at[pl.ds(off, CH)]], bt, st),
        )

    for b, (bw, bp, bt, ow, sw, sp, st, so) in enumerate(slots):
        for d in gathers(b * CH, bw, bp, bt, sw, sp, st):
            d.start()

    zero = jnp.zeros((LANES,), jnp.float32)

    def pair_body(p, _):
        for b, (bw, bp, bt, ow, sw, sp, st, so) in enumerate(slots):
            c = 2 * p + b
            off = pl.multiple_of(c * CH, CH)
            for d in gathers(off, bw, bp, bt, sw, sp, st):
                d.wait()

            def p1_body(t, _1):
                a = [zero, zero, zero, zero]
                a2 = [zero, zero, zero, zero]
                for j in range(NVEC):
                    sl = pl.ds(j * LANES, LANES)
                    s = bw[t, sl] + bp[t, sl] + bt[t, sl]
                    bw[t, sl] = s
                    a[j & 3] = a[j & 3] + s
                    a2[j & 3] = a2[j & 3] + s * s
                acc = (a[0] + a[1]) + (a[2] + a[3])
                acc2 = (a2[0] + a2[1]) + (a2[2] + a2[3])
                mean = _sum_lanes(acc) * (1.0 / HIDDEN)
                var = _sum_lanes(acc2) * (1.0 / HIDDEN) - mean * mean
                rstd = _rsqrt_vec(var + LN_EPS)
                rs_buf[t] = rstd
                nsh_buf[t] = -(mean * rstd)
                return 0

            lax.fori_loop(0, CH, p1_body, 0)

            @pl.when(p > 0)
            def _wait_out():
                prev = pl.multiple_of((c - 2) * CH, CH)
                pltpu.make_async_copy(
                    ow, out_h.at[pl.ds(base + prev, CH)], so).wait()

            def p2_body(g, _2):
                t0 = g * TG
                rs = [rs_buf[t0 + k] for k in range(TG)]
                ns = [nsh_buf[t0 + k] for k in range(TG)]
                for j in range(NVEC):
                    sl = pl.ds(j * LANES, LANES)
                    gj = gv[sl]
                    bj = bv[sl]
                    for k in range(TG):
                        s = bw[t0 + k, sl]
                        ow[t0 + k, sl] = (s * rs[k] + ns[k]) * gj + bj
                return 0

            lax.fori_loop(0, CH // TG, p2_body, 0)

            @pl.when(p < NPAIR - 1)
            def _refill():
                nxt = pl.multiple_of((c + 2) * CH, CH)
                for d in gathers(nxt, bw, bp, bt, sw, sp, st):
                    d.start()

            pltpu.make_async_copy(
                ow, out_h.at[pl.ds(base + off, CH)], so).start()
        return 0

    lax.fori_loop(0, NPAIR, pair_body, 0)

    for b, (bw, bp, bt, ow, sw, sp, st, so) in enumerate(slots):
        off = (NCH - 2 + b) * CH
        pltpu.make_async_copy(ow, out_h.at[pl.ds(base + off, CH)], so).wait()


def kernel(input_ids, token_type_ids, position_ids, word_embeddings,
           position_embeddings, token_type_embeddings, ln_gamma, ln_beta):
    B, S = input_ids.shape
    ids = jnp.asarray(input_ids, jnp.int32).reshape(-1)
    tids = jnp.asarray(token_type_ids, jnp.int32).reshape(-1)
    pids = jnp.asarray(position_ids, jnp.int32).reshape(-1)
    out = _embed_ln(ids, pids, tids, word_embeddings, position_embeddings,
                    token_type_embeddings, ln_gamma, ln_beta)
    return out.reshape(B, S, HIDDEN)


# speedup vs baseline: 1.3555x; 1.0094x over previous
import functools

import jax
import jax.numpy as jnp
from jax import lax
from jax.experimental import pallas as pl
from jax.experimental.pallas import tpu as pltpu
from jax.experimental.pallas import tpu_sc as plsc

HIDDEN = 768
LANES = 16
NVEC = HIDDEN // LANES
TOK = 8192
NC = 2
NS = 16
NW = NC * NS
PER_W = TOK // NW
CH = 16
NCH = PER_W // CH
NPAIR = NCH // 2
TG = 4
LN_EPS = 1e-12

_GATHER_DNUMS = lax.GatherDimensionNumbers(
    offset_dims=(), collapsed_slice_dims=(0,), start_index_map=(0,))


def _shuffle(x, idx):
    return lax.gather(x, idx[:, None], _GATHER_DNUMS, slice_sizes=(1,),
                      mode=lax.GatherScatterMode.PROMISE_IN_BOUNDS)


def _sum_lanes(x):
    lanes = lax.iota(jnp.int32, LANES)
    for k in (8, 4, 2, 1):
        x = x + _shuffle(x, lanes ^ k)
    return x


def _rsqrt_vec(x):
    i = lax.bitcast_convert_type(x, jnp.int32)
    i = jnp.int32(0x5F3759DF) - lax.shift_right_logical(i, 1)
    y = lax.bitcast_convert_type(i, jnp.float32)
    for _ in range(3):
        y = y * (1.5 - 0.5 * x * y * y)
    return y


_mesh = plsc.VectorSubcoreMesh(core_axis_name="c", subcore_axis_name="s")


@functools.partial(
    pl.kernel,
    mesh=_mesh,
    out_type=jax.ShapeDtypeStruct((TOK, HIDDEN), jnp.float32),
    scratch_types=[
        pltpu.VMEM((PER_W,), jnp.int32),
        pltpu.VMEM((PER_W,), jnp.int32),
        pltpu.VMEM((PER_W,), jnp.int32),
        pltpu.VMEM((CH, HIDDEN), jnp.float32),
        pltpu.VMEM((CH, HIDDEN), jnp.float32),
        pltpu.VMEM((CH, HIDDEN), jnp.float32),
        pltpu.VMEM((CH, HIDDEN), jnp.float32),
        pltpu.VMEM((CH, HIDDEN), jnp.float32),
        pltpu.VMEM((CH, HIDDEN), jnp.float32),
        pltpu.VMEM((CH, HIDDEN), jnp.float32),
        pltpu.VMEM((CH, HIDDEN), jnp.float32),
        pltpu.VMEM((CH, LANES), jnp.float32),
        pltpu.VMEM((CH, LANES), jnp.float32),
        pltpu.VMEM((HIDDEN,), jnp.float32),
        pltpu.VMEM((HIDDEN,), jnp.float32),
        pltpu.SemaphoreType.DMA,
        pltpu.SemaphoreType.DMA,
        pltpu.SemaphoreType.DMA,
        pltpu.SemaphoreType.DMA,
        pltpu.SemaphoreType.DMA,
        pltpu.SemaphoreType.DMA,
        pltpu.SemaphoreType.DMA,
        pltpu.SemaphoreType.DMA,
    ],
)
def _embed_ln(ids_h, pids_h, tids_h, wtab_h, ptab_h, ttab_h, g_h, b_h,
              out_h, idw, idp, idt,
              bw0, bp0, bt0, bw1, bp1, bt1, ow0, ow1, rs_buf, nsh_buf,
              gv, bv, sw0, sp0, st0, sw1, sp1, st1, so0, so1):
    wid = lax.axis_index("s") * NC + lax.axis_index("c")
    base = wid * PER_W
    pltpu.sync_copy(ids_h.at[pl.ds(base, PER_W)], idw)
    pltpu.sync_copy(pids_h.at[pl.ds(base, PER_W)], idp)
    pltpu.sync_copy(tids_h.at[pl.ds(base, PER_W)], idt)
    pltpu.sync_copy(g_h, gv)
    pltpu.sync_copy(b_h, bv)

    slots = (
        (bw0, bp0, bt0, ow0, sw0, sp0, st0, so0),
        (bw1, bp1, bt1, ow1, sw1, sp1, st1, so1),
    )

    def gathers(off, bw, bp, bt, sw, sp, st):
        return (
            pltpu.make_async_copy(wtab_h.at[idw.at[pl.ds(off, CH)]], bw, sw),
            pltpu.make_async_copy(ptab_h.at[idp.at[pl.ds(off, CH)]], bp, sp),
            pltpu.make_async_copy(ttab_h.at[idt.at[pl.ds(off, CH)]], bt, st),
        )

    for b, (bw, bp, bt, ow, sw, sp, st, so) in enumerate(slots):
        for d in gathers(b * CH, bw, bp, bt, sw, sp, st):
            d.start()

    zero = jnp.zeros((LANES,), jnp.float32)

    def pair_body(p, _):
        for b, (bw, bp, bt, ow, sw, sp, st, so) in enumerate(slots):
            c = 2 * p + b
            off = pl.multiple_of(c * CH, CH)
            for d in gathers(off, bw, bp, bt, sw, sp, st):
                d.wait()

            @plsc.parallel_loop(0, CH, step=1, unroll=2)
            def p1_body(t):
                a = [zero, zero, zero, zero]
                a2 = [zero, zero, zero, zero]
                for j in range(NVEC):
                    sl = pl.ds(j * LANES, LANES)
                    s = bw[t, sl] + bp[t, sl] + bt[t, sl]
                    bw[t, sl] = s
                    a[j & 3] = a[j & 3] + s
                    a2[j & 3] = a2[j & 3] + s * s
                acc = (a[0] + a[1]) + (a[2] + a[3])
                acc2 = (a2[0] + a2[1]) + (a2[2] + a2[3])
                mean = _sum_lanes(acc) * (1.0 / HIDDEN)
                var = _sum_lanes(acc2) * (1.0 / HIDDEN) - mean * mean
                rstd = _rsqrt_vec(var + LN_EPS)
                rs_buf[t] = rstd
                nsh_buf[t] = -(mean * rstd)

            @pl.when(p > 0)
            def _wait_out():
                prev = pl.multiple_of((c - 2) * CH, CH)
                pltpu.make_async_copy(
                    ow, out_h.at[pl.ds(base + prev, CH)], so).wait()

            @plsc.parallel_loop(0, CH // TG, step=1, unroll=2)
            def p2_body(g):
                t0 = g * TG
                rs = [rs_buf[t0 + k] for k in range(TG)]
                ns = [nsh_buf[t0 + k] for k in range(TG)]
                for j in range(NVEC):
                    sl = pl.ds(j * LANES, LANES)
                    gj = gv[sl]
                    bj = bv[sl]
                    for k in range(TG):
                        s = bw[t0 + k, sl]
                        ow[t0 + k, sl] = (s * rs[k] + ns[k]) * gj + bj

            @pl.when(p < NPAIR - 1)
            def _refill():
                nxt = pl.multiple_of((c + 2) * CH, CH)
                for d in gathers(nxt, bw, bp, bt, sw, sp, st):
                    d.start()

            pltpu.make_async_copy(
                ow, out_h.at[pl.ds(base + off, CH)], so).start()
        return 0

    lax.fori_loop(0, NPAIR, pair_body, 0)

    for b, (bw, bp, bt, ow, sw, sp, st, so) in enumerate(slots):
        off = (NCH - 2 + b) * CH
        pltpu.make_async_copy(ow, out_h.at[pl.ds(base + off, CH)], so).wait()


def kernel(input_ids, token_type_ids, position_ids, word_embeddings,
           position_embeddings, token_type_embeddings, ln_gamma, ln_beta):
    B, S = input_ids.shape
    ids = jnp.asarray(input_ids, jnp.int32).reshape(-1)
    tids = jnp.asarray(token_type_ids, jnp.int32).reshape(-1)
    pids = jnp.asarray(position_ids, jnp.int32).reshape(-1)
    out = _embed_ln(ids, pids, tids, word_embeddings, position_embeddings,
                    token_type_embeddings, ln_gamma, ln_beta)
    return out.reshape(B, S, HIDDEN)
